# baseline (speedup 1.0000x reference)
import functools

import jax
import jax.numpy as jnp
from jax import lax
from jax.experimental import pallas as pl
from jax.experimental.pallas import tpu as pltpu
from jax.experimental.pallas import tpu_sc as plsc

B, V = 64, 100000
NEG_BIG = -3.0e38

SC_T0 = 397
SC_NT = 384
V_TC = SC_T0 * 128
V_SC_END = (SC_T0 + SC_NT) * 128


TCW = 4096
TC_FULL = -(-V_TC // TCW)
TC_STEPS = TC_FULL + 1
TAIL_BLK = V_SC_END // TCW


def _tc_kernel(x_ref, a_ref, m_out, s_out, d_out, n_out, m_sc, s_sc, d_sc, n_sc):
    i = pl.program_id(0)

    @pl.when(i == 0)
    def _init():
        m_sc[...] = jnp.full_like(m_sc, NEG_BIG)
        s_sc[...] = jnp.zeros_like(s_sc)
        d_sc[...] = jnp.zeros_like(d_sc)
        n_sc[...] = jnp.zeros_like(n_sc)

    x = x_ref[...]
    a = a_ref[...]
    blk = jnp.where(i == TC_STEPS - 1, TAIL_BLK, i)
    col = blk * TCW + lax.broadcasted_iota(jnp.int32, x.shape, 1)
    lo = jnp.where(i == TC_STEPS - 1, V_SC_END, 0)
    hi = jnp.where(i == TC_STEPS - 1, V, V_TC)
    mask = (col >= lo) & (col < hi)
    xm = jnp.where(mask, x, NEG_BIG)

    m_old = m_sc[...]
    m_new = jnp.maximum(m_old, jnp.max(xm, axis=1, keepdims=True))
    s_sc[...] = s_sc[...] * jnp.exp(m_old - m_new) + jnp.sum(
        jnp.exp(xm - m_new), axis=1, keepdims=True
    )
    m_sc[...] = m_new
    am = jnp.where(mask, a, 0.0)
    d_sc[...] += jnp.sum(am * x, axis=1, keepdims=True)
    n_sc[...] += jnp.sum(am, axis=1, keepdims=True)

    @pl.when(i == TC_STEPS - 1)
    def _fin():
        m_out[...] = m_sc[...]
        s_out[...] = s_sc[...]
        d_out[...] = d_sc[...]
        n_out[...] = n_sc[...]


def _tc_partials(logits, actions):
    def imap(i):
        return (0, jnp.where(i == TC_STEPS - 1, TAIL_BLK, i))

    return pl.pallas_call(
        _tc_kernel,
        grid=(TC_STEPS,),
        in_specs=[
            pl.BlockSpec((B, TCW), imap),
            pl.BlockSpec((B, TCW), imap),
        ],
        out_specs=[pl.BlockSpec((B, 1), lambda i: (0, 0))] * 4,
        out_shape=[jax.ShapeDtypeStruct((B, 1), jnp.float32)] * 4,
        scratch_shapes=[pltpu.VMEM((B, 1), jnp.float32)] * 4,
    )(logits, actions)


NC, NS, L = 2, 16, 16
NW = NC * NS
TR = 8
CS = NW // TR
TPW = SC_NT // CS
CT = 16
NCH = TPW // CT
CW = CT * 128
NBUF = 3


@functools.partial(
    pl.kernel,
    out_type=[jax.ShapeDtypeStruct((NW * TR, L), jnp.float32)] * 4,
    mesh=plsc.VectorSubcoreMesh(core_axis_name="c", subcore_axis_name="s"),
    compiler_params=pltpu.CompilerParams(use_tc_tiling_on_sc=True),
    scratch_types=[
        pltpu.VMEM((TR, CW), jnp.float32),
        pltpu.VMEM((TR, CW), jnp.float32),
        pltpu.VMEM((TR, CW), jnp.float32),
        pltpu.VMEM((TR, CW), jnp.float32),
        pltpu.VMEM((TR, CW), jnp.float32),
        pltpu.VMEM((TR, CW), jnp.float32),
        pltpu.VMEM((TR, L), jnp.float32),
        pltpu.SemaphoreType.DMA,
        pltpu.SemaphoreType.DMA,
        pltpu.SemaphoreType.DMA,
        pltpu.SemaphoreType.DMA,
        pltpu.SemaphoreType.DMA,
        pltpu.SemaphoreType.DMA,
    ],
)
def _sc_partials(
    log_hbm, act_hbm, m_out, s_out, d_out, n_out,
    xb0, xb1, xb2, ab0, ab1, ab2, stage, sx0, sx1, sx2, sa0, sa1, sa2,
):
    wid = lax.axis_index("s") * NC + lax.axis_index("c")
    rtile = wid % TR
    cslab = wid // TR
    row0 = rtile * TR
    xbufs, abufs = (xb0, xb1, xb2), (ab0, ab1, ab2)
    sxs, sas = (sx0, sx1, sx2), (sa0, sa1, sa2)

    def start(k):
        col0 = (SC_T0 + cslab * TPW + k * CT) * 128
        cx = pltpu.async_copy(
            log_hbm.at[pl.ds(row0, TR), pl.ds(col0, CW)], xbufs[k % NBUF], sxs[k % NBUF]
        )
        ca = pltpu.async_copy(
            act_hbm.at[pl.ds(row0, TR), pl.ds(col0, CW)], abufs[k % NBUF], sas[k % NBUF]
        )
        return cx, ca

    m8 = [jnp.full((L,), NEG_BIG, jnp.float32) for _ in range(TR)]
    s8 = [jnp.zeros((L,), jnp.float32) for _ in range(TR)]
    d8 = [jnp.zeros((L,), jnp.float32) for _ in range(TR)]
    n8 = [jnp.zeros((L,), jnp.float32) for _ in range(TR)]

    ring = [start(0), start(1)]
    for k in range(NCH):
        if k + 2 < NCH:
            ring.append(start(k + 2))
        cur = ring.pop(0)
        cur[0].wait()
        cur[1].wait()
        xb, ab = xbufs[k % NBUF], abufs[k % NBUF]

        def maxbody(j, carry, xb=xb):
            return tuple(
                jnp.maximum(carry[r], xb[r, pl.ds(j * L, L)]) for r in range(TR)
            )

        m8_new = list(lax.fori_loop(0, CW // L, maxbody, tuple(m8)))
        for r in range(TR):
            s8[r] = s8[r] * jnp.exp(m8[r] - m8_new[r])
        m8 = m8_new

        def accbody(j, carry, xb=xb, ab=ab, m8=tuple(m8)):
            s, d, n = carry
            s, d, n = list(s), list(d), list(n)
            for r in range(TR):
                x = xb[r, pl.ds(j * L, L)]
                a = ab[r, pl.ds(j * L, L)]
                s[r] = s[r] + jnp.exp(x - m8[r])
                d[r] = d[r] + a * x
                n[r] = n[r] + a
            return tuple(s), tuple(d), tuple(n)

        s8, d8, n8 = lax.fori_loop(
            0, CW // L, accbody, (tuple(s8), tuple(d8), tuple(n8))
        )
        s8, d8, n8 = list(s8), list(d8), list(n8)

    for vecs, out in ((m8, m_out), (s8, s_out), (d8, d_out), (n8, n_out)):
        for r in range(TR):
            stage[r, pl.ds(0, L)] = vecs[r]
        pltpu.sync_copy(stage, out.at[pl.ds(wid * TR, TR), pl.ds(0, L)])


def _merge_kernel(
    m4_ref, s4_ref, d4_ref, n4_ref, mt_ref, st_ref, dt_ref, nt_ref, a01_ref, o_ref
):
    m_all = mt_ref[...]
    mp = []
    for c in range(CS):
        p = m4_ref[pl.ds(c * B, B), :]
        mp.append(p)
        m_all = jnp.maximum(m_all, jnp.max(p, axis=1, keepdims=True))
    s_all = st_ref[...] * jnp.exp(mt_ref[...] - m_all)
    d = dt_ref[...]
    n = nt_ref[...]
    for c in range(CS):
        s_all += jnp.sum(
            s4_ref[pl.ds(c * B, B), :] * jnp.exp(mp[c] - m_all),
            axis=1, keepdims=True,
        )
        d += jnp.sum(d4_ref[pl.ds(c * B, B), :], axis=1, keepdims=True)
        n += jnp.sum(n4_ref[pl.ds(c * B, B), :], axis=1, keepdims=True)
    lse = m_all + jnp.log(s_all)
    a0 = a01_ref[0]
    a1 = a01_ref[1]
    o_ref[...] = a1 + d - n * lse - (a0 * (V - n) + a1 * n)


def _merge(m4, s4, d4, n4, m_tc, s_tc, d_tc, n_tc, a01):
    return pl.pallas_call(
        _merge_kernel,
        in_specs=[pl.BlockSpec(memory_space=pltpu.VMEM)] * 8
        + [pl.BlockSpec(memory_space=pltpu.SMEM)],
        out_specs=pl.BlockSpec(memory_space=pltpu.VMEM),
        out_shape=jax.ShapeDtypeStruct((B, 1), jnp.float32),
    )(m4, s4, d4, n4, m_tc, s_tc, d_tc, n_tc, a01)


def kernel(logits, actions):
    from jax.scipy.special import gammaln

    m_tc, s_tc, d_tc, n_tc = _tc_partials(logits, actions)
    m4, s4, d4, n4 = _sc_partials(logits, actions)
    rt_zero = jnp.minimum(s_tc[0, 0], jnp.float32(0.0))
    a01 = gammaln(jnp.stack([1.0 + rt_zero, 2.0 + rt_zero]).astype(jnp.float32))
    return _merge(m4, s4, d4, n4, m_tc, s_tc, d_tc, n_tc, a01)


# speedup vs baseline: 1.0574x; 1.0574x over previous
import functools

import jax
import jax.numpy as jnp
from jax import lax
from jax.experimental import pallas as pl
from jax.experimental.pallas import tpu as pltpu
from jax.experimental.pallas import tpu_sc as plsc

B, V = 64, 100000
NEG_BIG = -3.0e38

SC_T0 = 445
SC_NT = 336
V_TC = SC_T0 * 128
V_SC_END = (SC_T0 + SC_NT) * 128


TCW = 8192
TC_FULL = -(-V_TC // TCW)
TC_STEPS = TC_FULL + 1
TAIL_BLK = V_SC_END // TCW


def _tc_kernel(x_ref, a_ref, m_out, s_out, d_out, n_out, m_sc, s_sc, d_sc, n_sc):
    i = pl.program_id(0)

    @pl.when(i == 0)
    def _init():
        m_sc[...] = jnp.full_like(m_sc, NEG_BIG)
        s_sc[...] = jnp.zeros_like(s_sc)
        d_sc[...] = jnp.zeros_like(d_sc)
        n_sc[...] = jnp.zeros_like(n_sc)

    x = x_ref[...]
    a = a_ref[...]
    blk = jnp.where(i == TC_STEPS - 1, TAIL_BLK, i)
    col = blk * TCW + lax.broadcasted_iota(jnp.int32, x.shape, 1)
    lo = jnp.where(i == TC_STEPS - 1, V_SC_END, 0)
    hi = jnp.where(i == TC_STEPS - 1, V, V_TC)
    mask = (col >= lo) & (col < hi)
    xm = jnp.where(mask, x, NEG_BIG)

    m_old = m_sc[...]
    m_new = jnp.maximum(m_old, jnp.max(xm, axis=1, keepdims=True))
    s_sc[...] = s_sc[...] * jnp.exp(m_old - m_new) + jnp.sum(
        jnp.exp(xm - m_new), axis=1, keepdims=True
    )
    m_sc[...] = m_new
    am = jnp.where(mask, a, 0.0)
    d_sc[...] += jnp.sum(am * x, axis=1, keepdims=True)
    n_sc[...] += jnp.sum(am, axis=1, keepdims=True)

    @pl.when(i == TC_STEPS - 1)
    def _fin():
        m_out[...] = m_sc[...]
        s_out[...] = s_sc[...]
        d_out[...] = d_sc[...]
        n_out[...] = n_sc[...]


def _tc_partials(logits, actions):
    def imap(i):
        return (0, jnp.where(i == TC_STEPS - 1, TAIL_BLK, i))

    return pl.pallas_call(
        _tc_kernel,
        grid=(TC_STEPS,),
        in_specs=[
            pl.BlockSpec((B, TCW), imap),
            pl.BlockSpec((B, TCW), imap),
        ],
        out_specs=[pl.BlockSpec((B, 1), lambda i: (0, 0))] * 4,
        out_shape=[jax.ShapeDtypeStruct((B, 1), jnp.float32)] * 4,
        scratch_shapes=[pltpu.VMEM((B, 1), jnp.float32)] * 4,
    )(logits, actions)


NC, NS, L = 2, 16, 16
NW = NC * NS
TR = 8
CS = NW // TR
TPW = SC_NT // CS
CT = 12
NCH = TPW // CT
CW = CT * 128
NBUF = 3


@functools.partial(
    pl.kernel,
    out_type=[jax.ShapeDtypeStruct((NW * TR, L), jnp.float32)] * 4,
    mesh=plsc.VectorSubcoreMesh(core_axis_name="c", subcore_axis_name="s"),
    compiler_params=pltpu.CompilerParams(use_tc_tiling_on_sc=True),
    scratch_types=[
        pltpu.VMEM((TR, CW), jnp.float32),
        pltpu.VMEM((TR, CW), jnp.float32),
        pltpu.VMEM((TR, CW), jnp.float32),
        pltpu.VMEM((TR, CW), jnp.float32),
        pltpu.VMEM((TR, CW), jnp.float32),
        pltpu.VMEM((TR, CW), jnp.float32),
        pltpu.VMEM((TR, L), jnp.float32),
        pltpu.SemaphoreType.DMA,
        pltpu.SemaphoreType.DMA,
        pltpu.SemaphoreType.DMA,
        pltpu.SemaphoreType.DMA,
        pltpu.SemaphoreType.DMA,
        pltpu.SemaphoreType.DMA,
    ],
)
def _sc_partials(
    log_hbm, act_hbm, m_out, s_out, d_out, n_out,
    xb0, xb1, xb2, ab0, ab1, ab2, stage, sx0, sx1, sx2, sa0, sa1, sa2,
):
    wid = lax.axis_index("s") * NC + lax.axis_index("c")
    rtile = wid % TR
    cslab = wid // TR
    row0 = rtile * TR
    xbufs, abufs = (xb0, xb1, xb2), (ab0, ab1, ab2)
    sxs, sas = (sx0, sx1, sx2), (sa0, sa1, sa2)

    def start(k):
        col0 = (SC_T0 + cslab * TPW + k * CT) * 128
        cx = pltpu.async_copy(
            log_hbm.at[pl.ds(row0, TR), pl.ds(col0, CW)], xbufs[k % NBUF], sxs[k % NBUF]
        )
        ca = pltpu.async_copy(
            act_hbm.at[pl.ds(row0, TR), pl.ds(col0, CW)], abufs[k % NBUF], sas[k % NBUF]
        )
        return cx, ca

    m8 = [jnp.full((L,), NEG_BIG, jnp.float32) for _ in range(TR)]
    s8 = [jnp.zeros((L,), jnp.float32) for _ in range(TR)]
    d8 = [jnp.zeros((L,), jnp.float32) for _ in range(TR)]
    n8 = [jnp.zeros((L,), jnp.float32) for _ in range(TR)]

    ring = [start(0), start(1)]
    for k in range(NCH):
        if k + 2 < NCH:
            ring.append(start(k + 2))
        cur = ring.pop(0)
        cur[0].wait()
        cur[1].wait()
        xb, ab = xbufs[k % NBUF], abufs[k % NBUF]

        def maxbody(j, carry, xb=xb):
            return tuple(
                jnp.maximum(carry[r], xb[r, pl.ds(j * L, L)]) for r in range(TR)
            )

        m8_new = list(lax.fori_loop(0, CW // L, maxbody, tuple(m8)))
        for r in range(TR):
            s8[r] = s8[r] * jnp.exp(m8[r] - m8_new[r])
        m8 = m8_new

        def accbody(j, carry, xb=xb, ab=ab, m8=tuple(m8)):
            s, d, n = carry
            s, d, n = list(s), list(d), list(n)
            for r in range(TR):
                x = xb[r, pl.ds(j * L, L)]
                a = ab[r, pl.ds(j * L, L)]
                s[r] = s[r] + jnp.exp(x - m8[r])
                d[r] = d[r] + a * x
                n[r] = n[r] + a
            return tuple(s), tuple(d), tuple(n)

        s8, d8, n8 = lax.fori_loop(
            0, CW // L, accbody, (tuple(s8), tuple(d8), tuple(n8))
        )
        s8, d8, n8 = list(s8), list(d8), list(n8)

    for vecs, out in ((m8, m_out), (s8, s_out), (d8, d_out), (n8, n_out)):
        for r in range(TR):
            stage[r, pl.ds(0, L)] = vecs[r]
        pltpu.sync_copy(stage, out.at[pl.ds(wid * TR, TR), pl.ds(0, L)])


def _merge_kernel(
    m4_ref, s4_ref, d4_ref, n4_ref, mt_ref, st_ref, dt_ref, nt_ref, a01_ref, o_ref
):
    m_all = mt_ref[...]
    mp = []
    for c in range(CS):
        p = m4_ref[pl.ds(c * B, B), :]
        mp.append(p)
        m_all = jnp.maximum(m_all, jnp.max(p, axis=1, keepdims=True))
    s_all = st_ref[...] * jnp.exp(mt_ref[...] - m_all)
    d = dt_ref[...]
    n = nt_ref[...]
    for c in range(CS):
        s_all += jnp.sum(
            s4_ref[pl.ds(c * B, B), :] * jnp.exp(mp[c] - m_all),
            axis=1, keepdims=True,
        )
        d += jnp.sum(d4_ref[pl.ds(c * B, B), :], axis=1, keepdims=True)
        n += jnp.sum(n4_ref[pl.ds(c * B, B), :], axis=1, keepdims=True)
    lse = m_all + jnp.log(s_all)
    a0 = a01_ref[0]
    a1 = a01_ref[1]
    o_ref[...] = a1 + d - n * lse - (a0 * (V - n) + a1 * n)


def _merge(m4, s4, d4, n4, m_tc, s_tc, d_tc, n_tc, a01):
    return pl.pallas_call(
        _merge_kernel,
        in_specs=[pl.BlockSpec(memory_space=pltpu.VMEM)] * 8
        + [pl.BlockSpec(memory_space=pltpu.SMEM)],
        out_specs=pl.BlockSpec(memory_space=pltpu.VMEM),
        out_shape=jax.ShapeDtypeStruct((B, 1), jnp.float32),
    )(m4, s4, d4, n4, m_tc, s_tc, d_tc, n_tc, a01)


def kernel(logits, actions):
    from jax.scipy.special import gammaln

    m_tc, s_tc, d_tc, n_tc = _tc_partials(logits, actions)
    m4, s4, d4, n4 = _sc_partials(logits, actions)
    rt_zero = jnp.minimum(s_tc[0, 0], jnp.float32(0.0))
    a01 = gammaln(jnp.stack([1.0 + rt_zero, 2.0 + rt_zero]).astype(jnp.float32))
    return _merge(m4, s4, d4, n4, m_tc, s_tc, d_tc, n_tc, a01)


# speedup vs baseline: 1.0599x; 1.0024x over previous
import functools

import jax
import jax.numpy as jnp
from jax import lax
from jax.experimental import pallas as pl
from jax.experimental.pallas import tpu as pltpu
from jax.experimental.pallas import tpu_sc as plsc

B, V = 64, 100000
NEG_BIG = -3.0e38

SC_T0 = 480
SC_NT = 300
V_TC = SC_T0 * 128
V_SC_END = (SC_T0 + SC_NT) * 128


TCW = 7680
TC_STEPS = V_TC // TCW


def _tc_kernel(x_ref, a_ref, m_out, s_out, d_out, n_out, m_sc, s_sc, d_sc, n_sc):
    i = pl.program_id(0)

    @pl.when(i == 0)
    def _init():
        m_sc[...] = jnp.full_like(m_sc, NEG_BIG)
        s_sc[...] = jnp.zeros_like(s_sc)
        d_sc[...] = jnp.zeros_like(d_sc)
        n_sc[...] = jnp.zeros_like(n_sc)

    x = x_ref[...]
    a = a_ref[...]
    m_old = m_sc[...]
    m_new = jnp.maximum(m_old, jnp.max(x, axis=1, keepdims=True))
    s_sc[...] = s_sc[...] * jnp.exp(m_old - m_new) + jnp.sum(
        jnp.exp(x - m_new), axis=1, keepdims=True
    )
    m_sc[...] = m_new
    d_sc[...] += jnp.sum(a * x, axis=1, keepdims=True)
    n_sc[...] += jnp.sum(a, axis=1, keepdims=True)

    @pl.when(i == TC_STEPS - 1)
    def _fin():
        m_out[...] = m_sc[...]
        s_out[...] = s_sc[...]
        d_out[...] = d_sc[...]
        n_out[...] = n_sc[...]


def _tc_partials(logits, actions):
    return pl.pallas_call(
        _tc_kernel,
        grid=(TC_STEPS,),
        in_specs=[
            pl.BlockSpec((B, TCW), lambda i: (0, i)),
            pl.BlockSpec((B, TCW), lambda i: (0, i)),
        ],
        out_specs=[pl.BlockSpec((B, 1), lambda i: (0, 0))] * 4,
        out_shape=[jax.ShapeDtypeStruct((B, 1), jnp.float32)] * 4,
        scratch_shapes=[pltpu.VMEM((B, 1), jnp.float32)] * 4,
    )(logits, actions)


NC, NS, L = 2, 16, 16
NW = NC * NS
TR = 8
CS = NW // TR
TPW = SC_NT // CS
CT = 15
NCH = TPW // CT
CW = CT * 128
NBUF = 3


@functools.partial(
    pl.kernel,
    out_type=[jax.ShapeDtypeStruct((NW * TR, L), jnp.float32)] * 4,
    mesh=plsc.VectorSubcoreMesh(core_axis_name="c", subcore_axis_name="s"),
    compiler_params=pltpu.CompilerParams(use_tc_tiling_on_sc=True),
    scratch_types=[
        pltpu.VMEM((TR, CW), jnp.float32),
        pltpu.VMEM((TR, CW), jnp.float32),
        pltpu.VMEM((TR, CW), jnp.float32),
        pltpu.VMEM((TR, CW), jnp.float32),
        pltpu.VMEM((TR, CW), jnp.float32),
        pltpu.VMEM((TR, CW), jnp.float32),
        pltpu.VMEM((TR, L), jnp.float32),
        pltpu.SemaphoreType.DMA,
        pltpu.SemaphoreType.DMA,
        pltpu.SemaphoreType.DMA,
        pltpu.SemaphoreType.DMA,
        pltpu.SemaphoreType.DMA,
        pltpu.SemaphoreType.DMA,
    ],
)
def _sc_partials(
    log_hbm, act_hbm, m_out, s_out, d_out, n_out,
    xb0, xb1, xb2, ab0, ab1, ab2, stage, sx0, sx1, sx2, sa0, sa1, sa2,
):
    wid = lax.axis_index("s") * NC + lax.axis_index("c")
    rtile = wid % TR
    cslab = wid // TR
    row0 = rtile * TR
    xbufs, abufs = (xb0, xb1, xb2), (ab0, ab1, ab2)
    sxs, sas = (sx0, sx1, sx2), (sa0, sa1, sa2)

    def start(k):
        col0 = (SC_T0 + cslab * TPW + k * CT) * 128
        cx = pltpu.async_copy(
            log_hbm.at[pl.ds(row0, TR), pl.ds(col0, CW)], xbufs[k % NBUF], sxs[k % NBUF]
        )
        ca = pltpu.async_copy(
            act_hbm.at[pl.ds(row0, TR), pl.ds(col0, CW)], abufs[k % NBUF], sas[k % NBUF]
        )
        return cx, ca

    m8 = [jnp.full((L,), NEG_BIG, jnp.float32) for _ in range(TR)]
    s8 = [jnp.zeros((L,), jnp.float32) for _ in range(TR)]
    d8 = [jnp.zeros((L,), jnp.float32) for _ in range(TR)]
    n8 = [jnp.zeros((L,), jnp.float32) for _ in range(TR)]

    ring = [start(0), start(1)]
    for k in range(NCH):
        if k + 2 < NCH:
            ring.append(start(k + 2))
        cur = ring.pop(0)
        cur[0].wait()
        cur[1].wait()
        xb, ab = xbufs[k % NBUF], abufs[k % NBUF]

        def maxbody(j, carry, xb=xb):
            return tuple(
                jnp.maximum(carry[r], xb[r, pl.ds(j * L, L)]) for r in range(TR)
            )

        m8_new = list(lax.fori_loop(0, CW // L, maxbody, tuple(m8)))
        for r in range(TR):
            s8[r] = s8[r] * jnp.exp(m8[r] - m8_new[r])
        m8 = m8_new

        def accbody(j, carry, xb=xb, ab=ab, m8=tuple(m8)):
            s, d, n = carry
            s, d, n = list(s), list(d), list(n)
            for r in range(TR):
                x = xb[r, pl.ds(j * L, L)]
                a = ab[r, pl.ds(j * L, L)]
                s[r] = s[r] + jnp.exp(x - m8[r])
                d[r] = d[r] + a * x
                n[r] = n[r] + a
            return tuple(s), tuple(d), tuple(n)

        s8, d8, n8 = lax.fori_loop(
            0, CW // L, accbody, (tuple(s8), tuple(d8), tuple(n8))
        )
        s8, d8, n8 = list(s8), list(d8), list(n8)

    for vecs, out in ((m8, m_out), (s8, s_out), (d8, d_out), (n8, n_out)):
        for r in range(TR):
            stage[r, pl.ds(0, L)] = vecs[r]
        pltpu.sync_copy(stage, out.at[pl.ds(wid * TR, TR), pl.ds(0, L)])


TAILW = 256
TAIL_BLK = V_SC_END // TAILW


def _merge_kernel(
    xt_ref, at_ref, m4_ref, s4_ref, d4_ref, n4_ref, mt_ref, st_ref, dt_ref,
    nt_ref, a01_ref, o_ref
):
    xt = xt_ref[...]
    at = at_ref[...]
    colmask = V_SC_END + lax.broadcasted_iota(jnp.int32, xt.shape, 1) < V
    xm = jnp.where(colmask, xt, NEG_BIG)
    am = jnp.where(colmask, at, 0.0)

    m_all = jnp.maximum(mt_ref[...], jnp.max(xm, axis=1, keepdims=True))
    mp = []
    for c in range(CS):
        p = m4_ref[pl.ds(c * B, B), :]
        mp.append(p)
        m_all = jnp.maximum(m_all, jnp.max(p, axis=1, keepdims=True))
    s_all = st_ref[...] * jnp.exp(mt_ref[...] - m_all) + jnp.sum(
        jnp.exp(xm - m_all), axis=1, keepdims=True
    )
    d = dt_ref[...] + jnp.sum(am * xt, axis=1, keepdims=True)
    n = nt_ref[...] + jnp.sum(am, axis=1, keepdims=True)
    for c in range(CS):
        s_all += jnp.sum(
            s4_ref[pl.ds(c * B, B), :] * jnp.exp(mp[c] - m_all),
            axis=1, keepdims=True,
        )
        d += jnp.sum(d4_ref[pl.ds(c * B, B), :], axis=1, keepdims=True)
        n += jnp.sum(n4_ref[pl.ds(c * B, B), :], axis=1, keepdims=True)
    lse = m_all + jnp.log(s_all)
    a0 = a01_ref[0]
    a1 = a01_ref[1]
    o_ref[...] = a1 + d - n * lse - (a0 * (V - n) + a1 * n)


def _merge(logits, actions, m4, s4, d4, n4, m_tc, s_tc, d_tc, n_tc, a01):
    return pl.pallas_call(
        _merge_kernel,
        grid=(1,),
        in_specs=[
            pl.BlockSpec((B, TAILW), lambda i: (0, TAIL_BLK)),
            pl.BlockSpec((B, TAILW), lambda i: (0, TAIL_BLK)),
        ]
        + [pl.BlockSpec(memory_space=pltpu.VMEM)] * 8
        + [pl.BlockSpec(memory_space=pltpu.SMEM)],
        out_specs=pl.BlockSpec((B, 1), lambda i: (0, 0)),
        out_shape=jax.ShapeDtypeStruct((B, 1), jnp.float32),
    )(logits, actions, m4, s4, d4, n4, m_tc, s_tc, d_tc, n_tc, a01)


def kernel(logits, actions):
    from jax.scipy.special import gammaln

    m_tc, s_tc, d_tc, n_tc = _tc_partials(logits, actions)
    m4, s4, d4, n4 = _sc_partials(logits, actions)
    rt_zero = jnp.minimum(s_tc[0, 0], jnp.float32(0.0))
    a01 = gammaln(jnp.stack([1.0 + rt_zero, 2.0 + rt_zero]).astype(jnp.float32))
    return _merge(logits, actions, m4, s4, d4, n4, m_tc, s_tc, d_tc, n_tc, a01)


# speedup vs baseline: 1.0815x; 1.0203x over previous
import functools

import jax
import jax.numpy as jnp
from jax import lax
from jax.experimental import pallas as pl
from jax.experimental.pallas import tpu as pltpu
from jax.experimental.pallas import tpu_sc as plsc

B, V = 64, 100000
NEG_BIG = -3.0e38

SC_T0 = 540
SC_NT = 240
V_TC = SC_T0 * 128
V_SC_END = (SC_T0 + SC_NT) * 128


TCW = 7680
TC_STEPS = V_TC // TCW


def _tc_kernel(x_ref, a_ref, m_out, s_out, d_out, n_out, m_sc, s_sc, d_sc, n_sc):
    i = pl.program_id(0)

    @pl.when(i == 0)
    def _init():
        m_sc[...] = jnp.full_like(m_sc, NEG_BIG)
        s_sc[...] = jnp.zeros_like(s_sc)
        d_sc[...] = jnp.zeros_like(d_sc)
        n_sc[...] = jnp.zeros_like(n_sc)

    x = x_ref[...]
    a = a_ref[...]
    m_old = m_sc[...]
    m_new = jnp.maximum(m_old, jnp.max(x, axis=1, keepdims=True))
    s_sc[...] = s_sc[...] * jnp.exp(m_old - m_new) + jnp.sum(
        jnp.exp(x - m_new), axis=1, keepdims=True
    )
    m_sc[...] = m_new
    d_sc[...] += jnp.sum(a * x, axis=1, keepdims=True)
    n_sc[...] += jnp.sum(a, axis=1, keepdims=True)

    @pl.when(i == TC_STEPS - 1)
    def _fin():
        m_out[...] = m_sc[...]
        s_out[...] = s_sc[...]
        d_out[...] = d_sc[...]
        n_out[...] = n_sc[...]


def _tc_partials(logits, actions):
    return pl.pallas_call(
        _tc_kernel,
        grid=(TC_STEPS,),
        in_specs=[
            pl.BlockSpec((B, TCW), lambda i: (0, i)),
            pl.BlockSpec((B, TCW), lambda i: (0, i)),
        ],
        out_specs=[pl.BlockSpec((B, 1), lambda i: (0, 0))] * 4,
        out_shape=[jax.ShapeDtypeStruct((B, 1), jnp.float32)] * 4,
        scratch_shapes=[pltpu.VMEM((B, 1), jnp.float32)] * 4,
    )(logits, actions)


NC, NS, L = 2, 16, 16
NW = NC * NS
TR = 8
CS = NW // TR
TPW = SC_NT // CS
CT = 15
NCH = TPW // CT
CW = CT * 128
NBUF = 3


@functools.partial(
    pl.kernel,
    out_type=[jax.ShapeDtypeStruct((NW * TR, L), jnp.float32)] * 4,
    mesh=plsc.VectorSubcoreMesh(core_axis_name="c", subcore_axis_name="s"),
    compiler_params=pltpu.CompilerParams(use_tc_tiling_on_sc=True),
    scratch_types=[
        pltpu.VMEM((TR, CW), jnp.float32),
        pltpu.VMEM((TR, CW), jnp.float32),
        pltpu.VMEM((TR, CW), jnp.float32),
        pltpu.VMEM((TR, CW), jnp.float32),
        pltpu.VMEM((TR, CW), jnp.float32),
        pltpu.VMEM((TR, CW), jnp.float32),
        pltpu.VMEM((TR, L), jnp.float32),
        pltpu.SemaphoreType.DMA,
        pltpu.SemaphoreType.DMA,
        pltpu.SemaphoreType.DMA,
        pltpu.SemaphoreType.DMA,
        pltpu.SemaphoreType.DMA,
        pltpu.SemaphoreType.DMA,
    ],
)
def _sc_partials(
    log_hbm, act_hbm, m_out, s_out, d_out, n_out,
    xb0, xb1, xb2, ab0, ab1, ab2, stage, sx0, sx1, sx2, sa0, sa1, sa2,
):
    wid = lax.axis_index("s") * NC + lax.axis_index("c")
    rtile = wid % TR
    cslab = wid // TR
    row0 = rtile * TR
    xbufs, abufs = (xb0, xb1, xb2), (ab0, ab1, ab2)
    sxs, sas = (sx0, sx1, sx2), (sa0, sa1, sa2)

    def start(k):
        col0 = (SC_T0 + cslab * TPW + k * CT) * 128
        cx = pltpu.async_copy(
            log_hbm.at[pl.ds(row0, TR), pl.ds(col0, CW)], xbufs[k % NBUF], sxs[k % NBUF]
        )
        ca = pltpu.async_copy(
            act_hbm.at[pl.ds(row0, TR), pl.ds(col0, CW)], abufs[k % NBUF], sas[k % NBUF]
        )
        return cx, ca

    m8 = [jnp.full((L,), NEG_BIG, jnp.float32) for _ in range(TR)]
    s8 = [jnp.zeros((L,), jnp.float32) for _ in range(TR)]
    d8 = [jnp.zeros((L,), jnp.float32) for _ in range(TR)]
    n8 = [jnp.zeros((L,), jnp.float32) for _ in range(TR)]

    ring = [start(0), start(1)]
    for k in range(NCH):
        if k + 2 < NCH:
            ring.append(start(k + 2))
        cur = ring.pop(0)
        cur[0].wait()
        cur[1].wait()
        xb, ab = xbufs[k % NBUF], abufs[k % NBUF]

        def maxbody(j, carry, xb=xb):
            return tuple(
                jnp.maximum(carry[r], xb[r, pl.ds(j * L, L)]) for r in range(TR)
            )

        m8_new = list(lax.fori_loop(0, CW // L, maxbody, tuple(m8)))
        for r in range(TR):
            s8[r] = s8[r] * jnp.exp(m8[r] - m8_new[r])
        m8 = m8_new

        def accbody(j, carry, xb=xb, ab=ab, m8=tuple(m8)):
            s, d, n = carry
            s, d, n = list(s), list(d), list(n)
            for r in range(TR):
                x = xb[r, pl.ds(j * L, L)]
                a = ab[r, pl.ds(j * L, L)]
                s[r] = s[r] + jnp.exp(x - m8[r])
                d[r] = d[r] + a * x
                n[r] = n[r] + a
            return tuple(s), tuple(d), tuple(n)

        s8, d8, n8 = lax.fori_loop(
            0, CW // L, accbody, (tuple(s8), tuple(d8), tuple(n8))
        )
        s8, d8, n8 = list(s8), list(d8), list(n8)

    for vecs, out in ((m8, m_out), (s8, s_out), (d8, d_out), (n8, n_out)):
        for r in range(TR):
            stage[r, pl.ds(0, L)] = vecs[r]
        pltpu.sync_copy(stage, out.at[pl.ds(wid * TR, TR), pl.ds(0, L)])


TAILW = 256
TAIL_BLK = V_SC_END // TAILW


def _merge_kernel(
    xt_ref, at_ref, m4_ref, s4_ref, d4_ref, n4_ref, mt_ref, st_ref, dt_ref,
    nt_ref, a01_ref, o_ref
):
    xt = xt_ref[...]
    at = at_ref[...]
    colmask = V_SC_END + lax.broadcasted_iota(jnp.int32, xt.shape, 1) < V
    xm = jnp.where(colmask, xt, NEG_BIG)
    am = jnp.where(colmask, at, 0.0)

    m_all = jnp.maximum(mt_ref[...], jnp.max(xm, axis=1, keepdims=True))
    mp = []
    for c in range(CS):
        p = m4_ref[pl.ds(c * B, B), :]
        mp.append(p)
        m_all = jnp.maximum(m_all, jnp.max(p, axis=1, keepdims=True))
    s_all = st_ref[...] * jnp.exp(mt_ref[...] - m_all) + jnp.sum(
        jnp.exp(xm - m_all), axis=1, keepdims=True
    )
    d = dt_ref[...] + jnp.sum(am * xt, axis=1, keepdims=True)
    n = nt_ref[...] + jnp.sum(am, axis=1, keepdims=True)
    for c in range(CS):
        s_all += jnp.sum(
            s4_ref[pl.ds(c * B, B), :] * jnp.exp(mp[c] - m_all),
            axis=1, keepdims=True,
        )
        d += jnp.sum(d4_ref[pl.ds(c * B, B), :], axis=1, keepdims=True)
        n += jnp.sum(n4_ref[pl.ds(c * B, B), :], axis=1, keepdims=True)
    lse = m_all + jnp.log(s_all)
    a0 = a01_ref[0]
    a1 = a01_ref[1]
    o_ref[...] = a1 + d - n * lse - (a0 * (V - n) + a1 * n)


def _merge(logits, actions, m4, s4, d4, n4, m_tc, s_tc, d_tc, n_tc, a01):
    return pl.pallas_call(
        _merge_kernel,
        grid=(1,),
        in_specs=[
            pl.BlockSpec((B, TAILW), lambda i: (0, TAIL_BLK)),
            pl.BlockSpec((B, TAILW), lambda i: (0, TAIL_BLK)),
        ]
        + [pl.BlockSpec(memory_space=pltpu.VMEM)] * 8
        + [pl.BlockSpec(memory_space=pltpu.SMEM)],
        out_specs=pl.BlockSpec((B, 1), lambda i: (0, 0)),
        out_shape=jax.ShapeDtypeStruct((B, 1), jnp.float32),
    )(logits, actions, m4, s4, d4, n4, m_tc, s_tc, d_tc, n_tc, a01)


def kernel(logits, actions):
    from jax.scipy.special import gammaln

    m_tc, s_tc, d_tc, n_tc = _tc_partials(logits, actions)
    m4, s4, d4, n4 = _sc_partials(logits, actions)
    rt_zero = jnp.minimum(s_tc[0, 0], jnp.float32(0.0))
    a01 = gammaln(jnp.stack([1.0 + rt_zero, 2.0 + rt_zero]).astype(jnp.float32))
    return _merge(logits, actions, m4, s4, d4, n4, m_tc, s_tc, d_tc, n_tc, a01)
